# Initial kernel scaffold; baseline (speedup 1.0000x reference)
import jax
import jax.numpy as jnp
from jax.experimental import pallas as pl


def kernel(x, edge_index, edge_weight, subG_node, W_conv, b_conv, W_pred, b_pred):
    raise NotImplementedError("write your pallas kernel here")


# speedup vs baseline: 28.6700x; 28.6700x over previous
import functools

import jax
import jax.numpy as jnp
from jax import lax
from jax.experimental import pallas as pl
from jax.experimental.pallas import tpu as pltpu
from jax.experimental.pallas import tpu_sc as plsc

N = 10000
E = 320000
D = 128
T = 4
S = 1024
P = 16
C = 32

NC = 2
NS = 16
H = C // NC
NP = 10240
ZROWS = NP // NS
EWP = 20480
EP = EWP * NS
KCH = 2048
NCHUNK = EWP // KCH
SGW = S // NS

BN = 500


def _lane_bcast(v, l):
    idx = jnp.full((16, 1), l, dtype=jnp.int32)
    dn = lax.GatherDimensionNumbers(
        offset_dims=(), collapsed_slice_dims=(0,), start_index_map=(0,))
    return lax.gather(v, idx, dn, (1,),
                      mode=lax.GatherScatterMode.PROMISE_IN_BOUNDS)


def _tc_body(x_ref, wc_ref, wp_ref, bc_ref, bp_ref, g_ref, cv_ref):
    xs = jnp.sum(x_ref[...], axis=1)
    w2 = jnp.dot(wc_ref[...], wp_ref[...],
                 preferred_element_type=jnp.float32,
                 precision=lax.Precision.HIGHEST)
    g = jnp.dot(xs, w2, preferred_element_type=jnp.float32,
                precision=lax.Precision.HIGHEST) * (1.0 / T)
    g_ref[...] = g.reshape(BN, NC, H)
    cv_ref[...] = (float(P) * jnp.dot(bc_ref[...], wp_ref[...],
                                      preferred_element_type=jnp.float32)
                   + bp_ref[...])


_tc_call = pl.pallas_call(
    _tc_body,
    grid=(N // BN,),
    in_specs=[
        pl.BlockSpec((BN, T, D), lambda i: (i, 0, 0)),
        pl.BlockSpec((D, D), lambda i: (0, 0)),
        pl.BlockSpec((D, C), lambda i: (0, 0)),
        pl.BlockSpec((1, D), lambda i: (0, 0)),
        pl.BlockSpec((1, C), lambda i: (0, 0)),
    ],
    out_specs=[
        pl.BlockSpec((BN, NC, H), lambda i: (i, 0, 0)),
        pl.BlockSpec((1, C), lambda i: (0, 0)),
    ],
    out_shape=[
        jax.ShapeDtypeStruct((N, NC, H), jnp.float32),
        jax.ShapeDtypeStruct((1, C), jnp.float32),
    ],
)


def _sc_body(gfull, gsrc, didx, wflat, sgidx, zz, cv, out,
             sidx_v, didx_v, w_v, rows_v, sg_v, prow_v, cv_v, out_v,
             agg, gsem, ssem):
    c = lax.axis_index("c")
    s = lax.axis_index("s")

    pltpu.sync_copy(zz, agg.at[pl.ds(s * ZROWS, ZROWS)])
    pltpu.sync_copy(cv.at[c], cv_v)
    plsc.subcore_barrier()

    def chunk(t, carry):
        row0 = s * (EWP // 128) + t * (KCH // 128)
        pltpu.sync_copy(gsrc.at[c, pl.ds(row0, KCH // 128)], sidx_v)
        pltpu.sync_copy(didx.at[pl.ds(row0, KCH // 128)], didx_v)
        pltpu.sync_copy(wflat.at[pl.ds(row0 * 128, KCH)], w_v)
        gds = [pltpu.async_copy(gfull.at[sidx_v.at[j]],
                                rows_v.at[pl.ds(j * 128, 128)], gsem)
               for j in range(KCH // 128)]
        for d in gds:
            d.wait()
        sds = []
        for j in range(KCH // 128):
            def scale(m, _, j=j):
                r0 = j * 128 + m * 16
                w16 = w_v[pl.ds(r0, 16)]
                for l in range(16):
                    rows_v[r0 + l] = rows_v[r0 + l] * _lane_bcast(w16, l)
                return _
            lax.fori_loop(0, 8, scale, 0)
            sds.append(pltpu.async_copy(rows_v.at[pl.ds(j * 128, 128)],
                                        agg.at[didx_v.at[j]], ssem, add=True))
        for d in sds:
            d.wait()
        return carry

    lax.fori_loop(0, NCHUNK, chunk, 0)
    plsc.subcore_barrier()

    pltpu.sync_copy(sgidx.at[pl.ds(s * (SGW * P // 128), SGW * P // 128)],
                    sg_v)
    pds = [pltpu.async_copy(agg.at[sg_v.at[j]],
                            prow_v.at[pl.ds(j * 128, 128)], gsem)
           for j in range(SGW * P // 128)]
    for d in pds:
        d.wait()
    cvec = cv_v[...]

    def pool_one(q, carry):
        base = q * P
        acc = cvec
        for k in range(P):
            acc = acc + prow_v[base + k]
        out_v[q] = acc
        return carry

    lax.fori_loop(0, SGW, pool_one, 0)
    pltpu.sync_copy(out_v, out.at[c, pl.ds(s * SGW, SGW)])


_sc_call = functools.partial(
    pl.kernel,
    out_type=jax.ShapeDtypeStruct((NC, S, H), jnp.float32),
    mesh=plsc.VectorSubcoreMesh(core_axis_name="c", subcore_axis_name="s",
                                num_cores=NC, num_subcores=NS),
    compiler_params=pltpu.CompilerParams(use_tc_tiling_on_sc=False),
    scratch_types=[
        pltpu.VMEM((KCH // 128, 128), jnp.int32),
        pltpu.VMEM((KCH // 128, 128), jnp.int32),
        pltpu.VMEM((KCH,), jnp.float32),
        pltpu.VMEM((KCH, H), jnp.float32),
        pltpu.VMEM((SGW * P // 128, 128), jnp.int32),
        pltpu.VMEM((SGW * P, H), jnp.float32),
        pltpu.VMEM((16,), jnp.float32),
        pltpu.VMEM((SGW, H), jnp.float32),
        pltpu.VMEM_SHARED((NP, H), jnp.float32),
        pltpu.SemaphoreType.DMA,
        pltpu.SemaphoreType.DMA,
    ],
)(_sc_body)


def kernel(x, edge_index, edge_weight, subG_node, W_conv, b_conv, W_pred,
           b_pred):
    g3, cv = _tc_call(x, W_conv, W_pred, b_conv.reshape(1, D),
                      b_pred.reshape(1, C))
    gfull = g3.reshape(NC * N, H)
    cv2 = cv.reshape(NC, H)

    src = edge_index[0].astype(jnp.int32)
    dst = edge_index[1].astype(jnp.int32)
    pad = EP - E
    srcp = jnp.concatenate([src, jnp.zeros((pad,), jnp.int32)])
    dstp = jnp.concatenate([dst, jnp.zeros((pad,), jnp.int32)])
    wpad = jnp.concatenate([edge_weight, jnp.zeros((pad,), jnp.float32)])
    gsrc = jnp.stack([2 * srcp, 2 * srcp + 1]).reshape(NC, EP // 128, 128)
    didx = dstp.reshape(EP // 128, 128)
    sg = subG_node.astype(jnp.int32).reshape(S * P // 128, 128)
    zz = jnp.zeros((ZROWS, H), jnp.float32)

    outsc = _sc_call(gfull, gsrc, didx, wpad, sg, zz, cv2)
    return outsc.transpose(1, 0, 2).reshape(S, C)


# speedup vs baseline: 30.4451x; 1.0619x over previous
import functools

import jax
import jax.numpy as jnp
from jax import lax
from jax.experimental import pallas as pl
from jax.experimental.pallas import tpu as pltpu
from jax.experimental.pallas import tpu_sc as plsc

N = 10000
E = 320000
D = 128
T = 4
S = 1024
P = 16
C = 32

NC = 2
NS = 16
H = C // NC
NP = 10240
ZROWS = NP // NS
EWP = 20480
EP = EWP * NS
KCH = 2048
NCHUNK = EWP // KCH
SGW = S // NS

BN = 1000


def _lane_bcast(v, l):
    idx = jnp.full((16, 1), l, dtype=jnp.int32)
    dn = lax.GatherDimensionNumbers(
        offset_dims=(), collapsed_slice_dims=(0,), start_index_map=(0,))
    return lax.gather(v, idx, dn, (1,),
                      mode=lax.GatherScatterMode.PROMISE_IN_BOUNDS)


def _tc_body(x_ref, wc_ref, wp_ref, bc_ref, bp_ref, g_ref, cv_ref):
    xs = jnp.sum(x_ref[...], axis=1)
    w2 = jnp.dot(wc_ref[...], wp_ref[...],
                 preferred_element_type=jnp.float32,
                 precision=lax.Precision.HIGHEST)
    g = jnp.dot(xs, w2, preferred_element_type=jnp.float32,
                precision=lax.Precision.HIGHEST) * (1.0 / T)
    g_ref[...] = g
    cv_ref[...] = (float(P) * jnp.dot(bc_ref[...], wp_ref[...],
                                      preferred_element_type=jnp.float32)
                   + bp_ref[...])


_tc_call = pl.pallas_call(
    _tc_body,
    grid=(N // BN,),
    in_specs=[
        pl.BlockSpec((BN, T, D), lambda i: (i, 0, 0)),
        pl.BlockSpec((D, D), lambda i: (0, 0)),
        pl.BlockSpec((D, C), lambda i: (0, 0)),
        pl.BlockSpec((1, D), lambda i: (0, 0)),
        pl.BlockSpec((1, C), lambda i: (0, 0)),
    ],
    out_specs=[
        pl.BlockSpec((BN, C), lambda i: (i, 0)),
        pl.BlockSpec((1, C), lambda i: (0, 0)),
    ],
    out_shape=[
        jax.ShapeDtypeStruct((N, C), jnp.float32),
        jax.ShapeDtypeStruct((1, C), jnp.float32),
    ],
)


def _sc_body(gfull, gsrc, didx, wflat, sgidx, cv, out,
             sidx_v, didx_v, w_v, rows_v, sg_v, prow_v, cv_v, out_v,
             agg, gsem, ssem):
    c = lax.axis_index("c")
    s = lax.axis_index("s")

    zv = jnp.zeros((16,), jnp.float32)

    def zrow(i, carry):
        rows_v[i] = zv
        return carry

    lax.fori_loop(0, ZROWS, zrow, 0)
    pltpu.sync_copy(rows_v.at[pl.ds(0, ZROWS)],
                    agg.at[pl.ds(s * ZROWS, ZROWS)])
    pltpu.sync_copy(cv.at[c], cv_v)
    plsc.subcore_barrier()

    def chunk(t, carry):
        row0 = s * (EWP // 128) + t * (KCH // 128)
        pltpu.sync_copy(gsrc.at[c, pl.ds(row0, KCH // 128)], sidx_v)
        pltpu.sync_copy(didx.at[pl.ds(row0, KCH // 128)], didx_v)
        pltpu.sync_copy(wflat.at[pl.ds(row0 * 128, KCH)], w_v)
        gds = [pltpu.async_copy(gfull.at[sidx_v.at[j]],
                                rows_v.at[pl.ds(j * 128, 128)], gsem)
               for j in range(KCH // 128)]
        for d in gds:
            d.wait()
        sds = []
        for j in range(KCH // 128):
            def scale(m, _, j=j):
                r0 = j * 128 + m * 16
                w16 = w_v[pl.ds(r0, 16)]
                for l in range(16):
                    rows_v[r0 + l] = rows_v[r0 + l] * _lane_bcast(w16, l)
                return _
            lax.fori_loop(0, 8, scale, 0)
            sds.append(pltpu.async_copy(rows_v.at[pl.ds(j * 128, 128)],
                                        agg.at[didx_v.at[j]], ssem, add=True))
        for d in sds:
            d.wait()
        return carry

    lax.fori_loop(0, NCHUNK, chunk, 0)
    plsc.subcore_barrier()

    pltpu.sync_copy(sgidx.at[pl.ds(s * (SGW * P // 128), SGW * P // 128)],
                    sg_v)
    pds = [pltpu.async_copy(agg.at[sg_v.at[j]],
                            prow_v.at[pl.ds(j * 128, 128)], gsem)
           for j in range(SGW * P // 128)]
    for d in pds:
        d.wait()
    cvec = cv_v[...]

    def pool_one(q, carry):
        base = q * P
        acc = cvec
        for k in range(P):
            acc = acc + prow_v[base + k]
        out_v[q] = acc
        return carry

    lax.fori_loop(0, SGW, pool_one, 0)
    pltpu.sync_copy(out_v, out.at[c, pl.ds(s * SGW, SGW)])


_sc_call = functools.partial(
    pl.kernel,
    out_type=jax.ShapeDtypeStruct((NC, S, H), jnp.float32),
    mesh=plsc.VectorSubcoreMesh(core_axis_name="c", subcore_axis_name="s",
                                num_cores=NC, num_subcores=NS),
    compiler_params=pltpu.CompilerParams(use_tc_tiling_on_sc=False),
    scratch_types=[
        pltpu.VMEM((KCH // 128, 128), jnp.int32),
        pltpu.VMEM((KCH // 128, 128), jnp.int32),
        pltpu.VMEM((KCH,), jnp.float32),
        pltpu.VMEM((KCH, H), jnp.float32),
        pltpu.VMEM((SGW * P // 128, 128), jnp.int32),
        pltpu.VMEM((SGW * P, H), jnp.float32),
        pltpu.VMEM((16,), jnp.float32),
        pltpu.VMEM((SGW, H), jnp.float32),
        pltpu.VMEM_SHARED((NP, H), jnp.float32),
        pltpu.SemaphoreType.DMA,
        pltpu.SemaphoreType.DMA,
    ],
)(_sc_body)


def kernel(x, edge_index, edge_weight, subG_node, W_conv, b_conv, W_pred,
           b_pred):
    g3, cv = _tc_call(x, W_conv, W_pred, b_conv.reshape(1, D),
                      b_pred.reshape(1, C))
    gfull = g3.reshape(NC * N, H)
    cv2 = cv.reshape(NC, H)

    src = edge_index[0].astype(jnp.int32)
    dst = edge_index[1].astype(jnp.int32)
    pad = EP - E
    srcp = jnp.concatenate([src, jnp.zeros((pad,), jnp.int32)])
    dstp = jnp.concatenate([dst, jnp.zeros((pad,), jnp.int32)])
    wpad = jnp.concatenate([edge_weight, jnp.zeros((pad,), jnp.float32)])
    gsrc = jnp.stack([2 * srcp, 2 * srcp + 1]).reshape(NC, EP // 128, 128)
    didx = dstp.reshape(EP // 128, 128)
    sg = subG_node.astype(jnp.int32).reshape(S * P // 128, 128)

    outsc = _sc_call(gfull, gsrc, didx, wpad, sg, cv2)
    return outsc.transpose(1, 0, 2).reshape(S, C)


# speedup vs baseline: 32.6064x; 1.0710x over previous
import functools

import jax
import jax.numpy as jnp
from jax import lax
from jax.experimental import pallas as pl
from jax.experimental.pallas import tpu as pltpu
from jax.experimental.pallas import tpu_sc as plsc

N = 10000
E = 320000
D = 128
T = 4
S = 1024
P = 16
C = 32

NC = 2
NS = 16
H = C // NC
NP = 10240
ZROWS = NP // NS
EWP = 20480
EP = EWP * NS
KCH = 2048
NCHUNK = EWP // KCH
SGW = S // NS

BN = 1000


def _lane_bcast(v, l):
    idx = jnp.full((16, 1), l, dtype=jnp.int32)
    dn = lax.GatherDimensionNumbers(
        offset_dims=(), collapsed_slice_dims=(0,), start_index_map=(0,))
    return lax.gather(v, idx, dn, (1,),
                      mode=lax.GatherScatterMode.PROMISE_IN_BOUNDS)


def _tc_body(x_ref, wc_ref, wp_ref, bc_ref, bp_ref, g_ref, cv_ref):
    xs = jnp.sum(x_ref[...], axis=1)
    w2 = jnp.dot(wc_ref[...], wp_ref[...],
                 preferred_element_type=jnp.float32,
                 precision=lax.Precision.HIGHEST)
    g = jnp.dot(xs, w2, preferred_element_type=jnp.float32,
                precision=lax.Precision.HIGHEST) * (1.0 / T)
    g_ref[...] = g
    cv_ref[...] = (float(P) * jnp.dot(bc_ref[...], wp_ref[...],
                                      preferred_element_type=jnp.float32)
                   + bp_ref[...])


_tc_call = pl.pallas_call(
    _tc_body,
    grid=(N // BN,),
    in_specs=[
        pl.BlockSpec((BN, T, D), lambda i: (i, 0, 0)),
        pl.BlockSpec((D, D), lambda i: (0, 0)),
        pl.BlockSpec((D, C), lambda i: (0, 0)),
        pl.BlockSpec((1, D), lambda i: (0, 0)),
        pl.BlockSpec((1, C), lambda i: (0, 0)),
    ],
    out_specs=[
        pl.BlockSpec((BN, C), lambda i: (i, 0)),
        pl.BlockSpec((1, C), lambda i: (0, 0)),
    ],
    out_shape=[
        jax.ShapeDtypeStruct((N, C), jnp.float32),
        jax.ShapeDtypeStruct((1, C), jnp.float32),
    ],
)


def _sc_body(gfull, gsrc, didx, wflat, sgidx, cv, out,
             sidxA, didxA, wA, rowsA, sidxB, didxB, wB, rowsB,
             sg_v, prow_v, cv_v, out_v, agg,
             isem, gsemA, ssemA, gsemB, ssemB):
    c = lax.axis_index("c")
    s = lax.axis_index("s")
    KR = KCH // 128

    bufA = (sidxA, didxA, wA, rowsA, gsemA, ssemA)
    bufB = (sidxB, didxB, wB, rowsB, gsemB, ssemB)

    zv = jnp.zeros((16,), jnp.float32)

    def zrow(i, carry):
        rowsA[i] = zv
        return carry

    lax.fori_loop(0, ZROWS, zrow, 0)
    pltpu.sync_copy(rowsA.at[pl.ds(0, ZROWS)],
                    agg.at[pl.ds(s * ZROWS, ZROWS)])
    pltpu.sync_copy(cv.at[c], cv_v)
    pltpu.sync_copy(sgidx.at[pl.ds(s * (SGW * P // 128), SGW * P // 128)],
                    sg_v)
    plsc.subcore_barrier()

    def load_idx(t, buf):
        sidx, didx_b, w_b = buf[0], buf[1], buf[2]
        row0 = s * (EWP // 128) + t * KR
        ds = [pltpu.async_copy(gsrc.at[c, pl.ds(row0, KR)], sidx, isem),
              pltpu.async_copy(didx.at[pl.ds(row0, KR)], didx_b, isem),
              pltpu.async_copy(wflat.at[pl.ds(row0 * 128, KCH)], w_b, isem)]
        for d in ds:
            d.wait()

    def fire_gathers(buf):
        sidx, rows, gsem = buf[0], buf[3], buf[4]
        for j in range(KR):
            pltpu.async_copy(gfull.at[sidx.at[j]],
                             rows.at[pl.ds(j * 128, 128)], gsem)

    def wait_gathers(buf):
        pltpu.make_async_copy(gfull.at[pl.ds(0, KCH)], buf[3], buf[4]).wait()

    def scale(buf):
        w_b, rows = buf[2], buf[3]

        def one(m, carry):
            r0 = m * 16
            w16 = w_b[pl.ds(r0, 16)]
            for l in range(16):
                rows[r0 + l] = rows[r0 + l] * _lane_bcast(w16, l)
            return carry

        lax.fori_loop(0, KCH // 16, one, 0)

    def fire_scatters(buf):
        didx_b, rows, ssem = buf[1], buf[3], buf[5]
        for j in range(KR):
            pltpu.async_copy(rows.at[pl.ds(j * 128, 128)],
                             agg.at[didx_b.at[j]], ssem, add=True)

    def drain_scatters(buf):
        pltpu.make_async_copy(gfull.at[pl.ds(0, KCH)],
                              agg.at[pl.ds(0, KCH)], buf[5]).wait()

    NPAIR = NCHUNK // 2
    load_idx(0, bufA)
    fire_gathers(bufA)

    def pair(t2, carry):
        a = 2 * t2
        wait_gathers(bufA)
        scale(bufA)
        fire_scatters(bufA)

        @pl.when(t2 > 0)
        def _():
            drain_scatters(bufB)

        load_idx(a + 1, bufB)
        fire_gathers(bufB)
        wait_gathers(bufB)
        scale(bufB)
        fire_scatters(bufB)
        drain_scatters(bufA)

        @pl.when(t2 < NPAIR - 1)
        def _():
            load_idx(a + 2, bufA)
            fire_gathers(bufA)

        return carry

    lax.fori_loop(0, NPAIR, pair, 0)
    drain_scatters(bufB)
    plsc.subcore_barrier()

    pds = [pltpu.async_copy(agg.at[sg_v.at[j]],
                            prow_v.at[pl.ds(j * 128, 128)], gsemA)
           for j in range(SGW * P // 128)]
    for d in pds:
        d.wait()
    cvec = cv_v[...]

    def pool_one(q, carry):
        base = q * P
        acc = cvec
        for k in range(P):
            acc = acc + prow_v[base + k]
        out_v[q] = acc
        return carry

    lax.fori_loop(0, SGW, pool_one, 0)
    pltpu.sync_copy(out_v, out.at[c, pl.ds(s * SGW, SGW)])


_sc_call = functools.partial(
    pl.kernel,
    out_type=jax.ShapeDtypeStruct((NC, S, H), jnp.float32),
    mesh=plsc.VectorSubcoreMesh(core_axis_name="c", subcore_axis_name="s",
                                num_cores=NC, num_subcores=NS),
    compiler_params=pltpu.CompilerParams(use_tc_tiling_on_sc=False),
    scratch_types=(
        [pltpu.VMEM((KCH // 128, 128), jnp.int32),
         pltpu.VMEM((KCH // 128, 128), jnp.int32),
         pltpu.VMEM((KCH,), jnp.float32),
         pltpu.VMEM((KCH, H), jnp.float32)] * 2 +
        [pltpu.VMEM((SGW * P // 128, 128), jnp.int32),
         pltpu.VMEM((SGW * P, H), jnp.float32),
         pltpu.VMEM((16,), jnp.float32),
         pltpu.VMEM((SGW, H), jnp.float32),
         pltpu.VMEM_SHARED((NP, H), jnp.float32)] +
        [pltpu.SemaphoreType.DMA] * 5
    ),
)(_sc_body)


def kernel(x, edge_index, edge_weight, subG_node, W_conv, b_conv, W_pred,
           b_pred):
    g3, cv = _tc_call(x, W_conv, W_pred, b_conv.reshape(1, D),
                      b_pred.reshape(1, C))
    gfull = g3.reshape(NC * N, H)
    cv2 = cv.reshape(NC, H)

    src = edge_index[0].astype(jnp.int32)
    dst = edge_index[1].astype(jnp.int32)
    pad = EP - E
    srcp = jnp.concatenate([src, jnp.zeros((pad,), jnp.int32)])
    dstp = jnp.concatenate([dst, jnp.zeros((pad,), jnp.int32)])
    wpad = jnp.concatenate([edge_weight, jnp.zeros((pad,), jnp.float32)])
    gsrc = jnp.stack([2 * srcp, 2 * srcp + 1]).reshape(NC, EP // 128, 128)
    didx = dstp.reshape(EP // 128, 128)
    sg = subG_node.astype(jnp.int32).reshape(S * P // 128, 128)

    outsc = _sc_call(gfull, gsrc, didx, wpad, sg, cv2)
    return outsc.transpose(1, 0, 2).reshape(S, C)
